# Initial kernel scaffold; baseline (speedup 1.0000x reference)
import jax
import jax.numpy as jnp
from jax.experimental import pallas as pl


def kernel(x, edge_indices, W1, a_src1, a_dst1, b1, W2, a_src2, a_dst2, b2, Wfc, bfc):
    raise NotImplementedError("write your pallas kernel here")


# speedup vs baseline: 27.1016x; 27.1016x over previous
import jax
import jax.numpy as jnp
from jax import lax
from jax.experimental import pallas as pl
from jax.experimental.pallas import tpu as pltpu
from jax.experimental.pallas import tpu_sc as plsc

N = 10000
NP = 10112
NPS = NP // 16
NPT = 10240
NPTS = 640
NPTK = NPT // 8
NPTSK = NPTS // 8
E = 320000
ESL = E + N
TILES = 32
CH = 128
K = 84
EP = TILES * CH * K
NEG_SLOPE = 0.2
PAD_ROWS = 112

f32 = jnp.float32
i32 = jnp.int32

def _sc_pass1(s_hbm, d_hbm, al_pk, den_out,
              sx, dx, ssh, dsh, arows, brows, wp, al_sh, den_sh, sem, semh):
  cid = lax.axis_index("c")
  sid = lax.axis_index("s")
  wid = sid * 2 + cid
  lane = lax.iota(i32, 16)
  lane_lt4 = lane < 4
  shift4 = jnp.minimum(lane + 4, 15)
  z16 = jnp.zeros((16,), f32)

  pltpu.sync_copy(al_pk.at[pl.ds(sid * NPTSK, NPTSK)],
                  al_sh.at[pl.ds(sid * NPTSK, NPTSK)])

  def zb(j, c):
    for sub in range(8):
      wp[j, pl.ds(sub * 16, 16)] = z16
    return c
  lax.fori_loop(0, CH, zb, 0)
  pltpu.sync_copy(wp.at[pl.ds(0, NPTSK)],
                  den_sh.at[pl.ds(sid * NPTSK, NPTSK)])
  plsc.subcore_barrier()

  def chunk(j, c):
    row0 = wid * K + j
    cd = pltpu.async_copy(d_hbm.at[pl.ds(row0, 1)], dx, semh)
    pltpu.sync_copy(s_hbm.at[pl.ds(row0, 1)], sx)
    for k in range(8):
      sl = pl.ds(k * 16, 16)
      ssh[0, sl] = lax.shift_right_logical(sx[0, sl], 3)
    ca = pltpu.async_copy(al_sh.at[ssh.at[0]], arows, sem)
    cd.wait()
    for k in range(8):
      sl = pl.ds(k * 16, 16)
      dsh[0, sl] = lax.shift_right_logical(dx[0, sl], 3)
    cb = pltpu.async_copy(al_sh.at[dsh.at[0]], brows, semh)
    ca.wait()
    cb.wait()
    for k in range(8):
      sv = sx[0, pl.ds(k * 16, 16)]
      dv = dx[0, pl.ds(k * 16, 16)]
      for l in range(16):
        e = k * 16 + l
        ssc = sv[l]
        dsc = dv[l]
        av = arows[e, pl.ds((ssc % 8) * 16, 16)]
        bv = brows[e, pl.ds((dsc % 8) * 16, 16)]
        a = av + jnp.take(bv, shift4)
        a = jnp.where(a >= 0.0, a, NEG_SLOPE * a)
        w = jnp.where(lane_lt4, jnp.exp(jnp.minimum(a, 60.0)), 0.0)
        dslot = dsc % 8
        for slot in range(8):
          wp[e, pl.ds(slot * 16, 16)] = jnp.where(dslot == slot, w, z16)
    pltpu.sync_copy(wp, den_sh.at[dsh.at[0]], add=True)
    return c

  lax.fori_loop(0, K, chunk, 0)
  plsc.subcore_barrier()
  pltpu.sync_copy(den_sh.at[pl.ds(sid * NPTSK, NPTSK)],
                  den_out.at[pl.ds(cid * NPTK + sid * NPTSK, NPTSK)])


def _make_pass1():
  mesh = plsc.VectorSubcoreMesh(core_axis_name="c", subcore_axis_name="s")
  return pl.kernel(
      _sc_pass1,
      out_type=pltpu.HBM((2 * NPTK, 128), f32),
      mesh=mesh,
      scratch_types=[
          pltpu.VMEM((1, 128), i32),
          pltpu.VMEM((1, 128), i32),
          pltpu.VMEM((1, 128), i32),
          pltpu.VMEM((1, 128), i32),
          pltpu.VMEM((CH, 128), f32),
          pltpu.VMEM((CH, 128), f32),
          pltpu.VMEM((CH, 128), f32),
          pltpu.VMEM_SHARED((NPTK, 128), f32),
          pltpu.VMEM_SHARED((NPTK, 128), f32),
          pltpu.SemaphoreType.DMA,
          pltpu.SemaphoreType.DMA,
      ],
  )


def _sc_pass2(s_hbm, d_hbm, tb_pk, h_hbm, out_hbm,
              sx, dx, ssh, dsh, arows, brows, hbuf, tb_sh, out_sh, sem, semh,
              semd):
  cid = lax.axis_index("c")
  sid = lax.axis_index("s")
  wid = sid * 2 + cid
  lane = lax.iota(i32, 16)
  shift4 = jnp.minimum(lane + 4, 15)
  shift8 = jnp.minimum(lane + 8, 15)
  z16 = jnp.zeros((16,), f32)

  pltpu.sync_copy(tb_pk.at[pl.ds(sid * NPTSK, NPTSK)],
                  tb_sh.at[pl.ds(sid * NPTSK, NPTSK)])

  def zb(j, c):
    for sub in range(8):
      hbuf[j, pl.ds(sub * 16, 16)] = z16
    return c
  lax.fori_loop(0, 64, zb, 0)
  for q in range(NPS // 64):
    pltpu.sync_copy(hbuf, out_sh.at[pl.ds(sid * NPS + q * 64, 64)])
  pltpu.sync_copy(hbuf.at[pl.ds(0, NPS % 64)],
                  out_sh.at[pl.ds(sid * NPS + (NPS // 64) * 64, NPS % 64)])
  plsc.subcore_barrier()

  def chunk(j, c):
    row0 = wid * (2 * K) + j
    cd = pltpu.async_copy(d_hbm.at[pl.ds(row0, 1)], dx, semd)
    pltpu.sync_copy(s_hbm.at[pl.ds(row0, 1)], sx)
    ch = pltpu.async_copy(h_hbm.at[sx.at[0]], hbuf, semh)
    for k in range(4):
      sl = pl.ds(k * 16, 16)
      ssh[0, sl] = lax.shift_right_logical(sx[0, sl], 3)
    ca = pltpu.async_copy(tb_sh.at[ssh.at[0]], arows, sem)
    cd.wait()
    for k in range(4):
      sl = pl.ds(k * 16, 16)
      dsh[0, sl] = lax.shift_right_logical(dx[0, sl], 3)
    cb = pltpu.async_copy(tb_sh.at[dsh.at[0]], brows, semd)
    ca.wait()
    cb.wait()
    ch.wait()
    for k in range(4):
      sv = sx[0, pl.ds(k * 16, 16)]
      dv = dx[0, pl.ds(k * 16, 16)]
      for l in range(16):
        e = k * 16 + l
        ssc = sv[l]
        dsc = dv[l]
        av = arows[e, pl.ds((ssc % 8) * 16, 16)]
        bv = brows[e, pl.ds((dsc % 8) * 16, 16)]
        a = av + jnp.take(bv, shift4)
        a = jnp.where(a >= 0.0, a, NEG_SLOPE * a)
        w = jnp.exp(jnp.minimum(a, 60.0))
        coef = w * jnp.take(bv, shift8)
        for h in range(4):
          cf = jnp.take(coef, jnp.full((16,), h, dtype=i32))
          for sub in range(2):
            sl2 = pl.ds(h * 32 + sub * 16, 16)
            hbuf[e, sl2] = hbuf[e, sl2] * cf
    pltpu.sync_copy(hbuf, out_sh.at[dx.at[0]], add=True)
    return c

  lax.fori_loop(0, 2 * K, chunk, 0)
  plsc.subcore_barrier()
  pltpu.sync_copy(out_sh.at[pl.ds(sid * NPS, NPS)],
                  out_hbm.at[pl.ds(cid * NP + sid * NPS, NPS)])


def _make_pass2():
  mesh = plsc.VectorSubcoreMesh(core_axis_name="c", subcore_axis_name="s")
  return pl.kernel(
      _sc_pass2,
      out_type=pltpu.HBM((2 * NP, 128), f32),
      mesh=mesh,
      scratch_types=[
          pltpu.VMEM((1, 64), i32),
          pltpu.VMEM((1, 64), i32),
          pltpu.VMEM((1, 64), i32),
          pltpu.VMEM((1, 64), i32),
          pltpu.VMEM((64, 128), f32),
          pltpu.VMEM((64, 128), f32),
          pltpu.VMEM((64, 128), f32),
          pltpu.VMEM_SHARED((NPTK, 128), f32),
          pltpu.VMEM_SHARED((NP, 128), f32),
          pltpu.SemaphoreType.DMA,
          pltpu.SemaphoreType.DMA,
          pltpu.SemaphoreType.DMA,
      ],
  )


def _tc_a_body(x_ref, w1_ref, a1_ref, wfc_ref, bfc_ref, h_ref, al_ref, xch_ref):
  xb = x_ref[...]
  hb = jnp.dot(xb, w1_ref[...], preferred_element_type=f32)
  h_ref[...] = hb
  al_ref[...] = jnp.dot(hb, a1_ref[...], preferred_element_type=f32)
  xch_ref[...] = jnp.dot(xb, wfc_ref[...], preferred_element_type=f32) + bfc_ref[...]


def _tc_rec_body(den_ref, rec_ref):
  d = den_ref[0] + den_ref[1]
  rec_ref[...] = 1.0 / (d + 1e-16)


def _tc_e_body(o_ref, b1_ref, w2_ref, a2_ref, h2_ref, al2_ref):
  f1 = o_ref[0] + o_ref[1] + b1_ref[...]
  h2 = jnp.dot(f1, w2_ref[...], preferred_element_type=f32)
  h2_ref[...] = h2
  al2_ref[...] = jnp.dot(h2, a2_ref[...], preferred_element_type=f32)


def _tc_i_body(o_ref, b2_ref, xch_ref, out_ref):
  s = o_ref[0] + o_ref[1]
  m = (s[:, 0:32] + s[:, 32:64] + s[:, 64:96] + s[:, 96:128]) * 0.25
  out_ref[...] = m + b2_ref[...] + xch_ref[...]


def _attn_mat(a_src, a_dst):
  A = jnp.zeros((128, 8), f32)
  for h in range(4):
    A = A.at[h * 32:(h + 1) * 32, h].set(a_src[h])
    A = A.at[h * 32:(h + 1) * 32, 4 + h].set(a_dst[h])
  return A


@jax.jit
def kernel(x, edge_indices, W1, a_src1, a_dst1, b1, W2, a_src2, a_dst2, b2,
           Wfc, bfc):
  ei = edge_indices[1]
  src, dst = ei[0], ei[1]
  loop = jnp.arange(N, dtype=i32)
  npad = EP - ESL
  pad_i = jnp.arange(npad, dtype=i32)
  spf = jnp.concatenate([src, loop, pad_i % PAD_ROWS])
  dpf = jnp.concatenate([dst, loop, N + (pad_i % PAD_ROWS)])
  sp = spf.reshape(EP // 128, 128)
  dp = dpf.reshape(EP // 128, 128)
  sp64 = spf.reshape(EP // 64, 64)
  dp64 = dpf.reshape(EP // 64, 64)

  A1 = _attn_mat(a_src1, a_dst1)
  A2 = _attn_mat(a_src2, a_dst2)

  def pack16(t16):
    return jnp.pad(t16, ((0, NPT - NP), (0, 0))).reshape(NPT // 8, 128)

  R = 1000
  h1, al1, xch = pl.pallas_call(
      _tc_a_body,
      grid=(N // R,),
      in_specs=[
          pl.BlockSpec((R, 128), lambda i: (i, 0)),
          pl.BlockSpec((128, 128), lambda i: (0, 0)),
          pl.BlockSpec((128, 8), lambda i: (0, 0)),
          pl.BlockSpec((128, 32), lambda i: (0, 0)),
          pl.BlockSpec((1, 32), lambda i: (0, 0)),
      ],
      out_specs=[
          pl.BlockSpec((R, 128), lambda i: (i, 0)),
          pl.BlockSpec((R, 8), lambda i: (i, 0)),
          pl.BlockSpec((R, 32), lambda i: (i, 0)),
      ],
      out_shape=[
          jax.ShapeDtypeStruct((N, 128), f32),
          jax.ShapeDtypeStruct((N, 8), f32),
          jax.ShapeDtypeStruct((N, 32), f32),
      ],
  )(x, W1, A1, Wfc, bfc.reshape(1, 32))

  al1p = jnp.pad(al1, ((0, NP - N), (0, 0)))

  R2 = 1264
  pass1 = _make_pass1()
  den1 = pass1(sp, dp, pack16(jnp.pad(al1p, ((0, 0), (0, 8)))))

  rec1 = pl.pallas_call(
      _tc_rec_body,
      grid=(NP // R2,),
      in_specs=[pl.BlockSpec((2, R2, 16), lambda i: (0, i, 0))],
      out_specs=pl.BlockSpec((R2, 16), lambda i: (i, 0)),
      out_shape=jax.ShapeDtypeStruct((NP, 16), f32),
  )(den1.reshape(2, NPT, 16)[:, :NP, :])

  tb1 = jnp.concatenate(
      [al1p, rec1[:, 0:4], jnp.zeros((NP, 4), f32)], axis=1)

  pass2a = _make_pass2()
  out1 = pass2a(sp64, dp64, pack16(tb1), h1)

  h2, al2 = pl.pallas_call(
      _tc_e_body,
      grid=(NP // R2,),
      in_specs=[
          pl.BlockSpec((2, R2, 128), lambda i: (0, i, 0)),
          pl.BlockSpec((1, 128), lambda i: (0, 0)),
          pl.BlockSpec((128, 128), lambda i: (0, 0)),
          pl.BlockSpec((128, 8), lambda i: (0, 0)),
      ],
      out_specs=[
          pl.BlockSpec((R2, 128), lambda i: (i, 0)),
          pl.BlockSpec((R2, 8), lambda i: (i, 0)),
      ],
      out_shape=[
          jax.ShapeDtypeStruct((NP, 128), f32),
          jax.ShapeDtypeStruct((NP, 8), f32),
      ],
  )(out1.reshape(2, NP, 128), b1.reshape(1, 128), W2, A2)

  pass1b = _make_pass1()
  den2 = pass1b(sp, dp, pack16(jnp.pad(al2, ((0, 0), (0, 8)))))

  rec2 = pl.pallas_call(
      _tc_rec_body,
      grid=(NP // R2,),
      in_specs=[pl.BlockSpec((2, R2, 16), lambda i: (0, i, 0))],
      out_specs=pl.BlockSpec((R2, 16), lambda i: (i, 0)),
      out_shape=jax.ShapeDtypeStruct((NP, 16), f32),
  )(den2.reshape(2, NPT, 16)[:, :NP, :])

  tb2 = jnp.concatenate(
      [al2, rec2[:, 0:4], jnp.zeros((NP, 4), f32)], axis=1)

  pass2b = _make_pass2()
  out2 = pass2b(sp64, dp64, pack16(tb2), h2)

  final = pl.pallas_call(
      _tc_i_body,
      grid=(N // R,),
      in_specs=[
          pl.BlockSpec((2, R, 128), lambda i: (0, i, 0)),
          pl.BlockSpec((1, 32), lambda i: (0, 0)),
          pl.BlockSpec((R, 32), lambda i: (i, 0)),
      ],
      out_specs=pl.BlockSpec((R, 32), lambda i: (i, 0)),
      out_shape=jax.ShapeDtypeStruct((N, 32), f32),
  )(out2.reshape(2, NP, 128), b2.reshape(1, 32), xch)

  return final
